# baseline (speedup 1.0000x reference)
import functools
import math

import jax
import jax.numpy as jnp
from jax import lax
from jax.experimental import pallas as pl
from jax.experimental.pallas import tpu as pltpu
from jax.experimental.pallas import tpu_sc as plsc

D_MODEL = 768
N_HEAD = 12
D_HEAD = 64
N_EXP = 64
S_LEN = 2048
S_BLK = 512
EMA_DECAY = 0.99

_NC = 2
_NS = 16
_NW = _NC * _NS
_TOK_W = S_LEN // _NW


def _proj_kernel(x_ref, wq_ref, wk_ref, wr_ref, qk_ref, gl_ref):
    f32 = jnp.float32
    x = x_ref[...]
    dims = (((1,), (1,)), ((), ()))
    qk_ref[:, 0:D_MODEL] = jax.lax.dot_general(
        x, wq_ref[...], dims, preferred_element_type=f32)
    qk_ref[:, D_MODEL:2 * D_MODEL] = jax.lax.dot_general(
        x, wk_ref[...], dims, preferred_element_type=f32)
    gl_ref[...] = jax.lax.dot_general(
        x, wr_ref[...], dims, preferred_element_type=f32)


def _lane_shuffle(vec, perm):
    dn = lax.GatherDimensionNumbers(offset_dims=(), collapsed_slice_dims=(0,),
                                    start_index_map=(0,))
    return lax.gather(vec, perm[:, None], dn, (1,),
                      mode=lax.GatherScatterMode.PROMISE_IN_BOUNDS)


def _route_sc(gl_hbm, idx_hbm, buf, idxb, sem):
    i32 = jnp.int32
    w = lax.axis_index("s") * _NC + lax.axis_index("c")
    pltpu.async_copy(gl_hbm.at[pl.ds(w * _TOK_W * D_MODEL, _TOK_W * D_MODEL)],
                     buf, sem).wait()
    lane = lax.broadcasted_iota(i32, (16,), 0)
    perms = [lane ^ k for k in (1, 2, 4, 8)]

    def argmax64(off):
        v0 = buf[pl.ds(off, 16)]
        v1 = buf[pl.ds(off + 16, 16)]
        v2 = buf[pl.ds(off + 32, 16)]
        v3 = buf[pl.ds(off + 48, 16)]
        m = jnp.maximum(jnp.maximum(v0, v1), jnp.maximum(v2, v3))
        for p in perms:
            m = jnp.maximum(m, _lane_shuffle(m, p))
        cand = jnp.minimum(
            jnp.minimum(jnp.where(v0 == m, lane, N_EXP),
                        jnp.where(v1 == m, lane + 16, N_EXP)),
            jnp.minimum(jnp.where(v2 == m, lane + 32, N_EXP),
                        jnp.where(v3 == m, lane + 48, N_EXP)))
        for p in perms:
            cand = jnp.minimum(cand, _lane_shuffle(cand, p))
        return cand

    for h in range(N_HEAD):
        for g in range(_TOK_W // 16):
            def body(tt, vec):
                am = argmax64((g * 16 + tt) * D_MODEL + h * N_EXP)
                return jnp.where(lane == tt, am, vec)

            vec = lax.fori_loop(0, 16, body, jnp.zeros((16,), i32))
            idxb[pl.ds(h * _TOK_W + g * 16, 16)] = vec
    for h in range(N_HEAD):
        pltpu.sync_copy(idxb.at[pl.ds(h * _TOK_W, _TOK_W)],
                        idx_hbm.at[pl.ds(h * S_LEN + w * _TOK_W, _TOK_W)])


def _head_kernel(x_ref, q_ref, k_ref, idx_ref, wv_ref, wo_ref,
                 ema_ref, y_ref, ema_out_ref):
    f32 = jnp.float32
    i = pl.program_id(0)
    odd = (i & 1) == 1
    wv = wv_ref[0]
    wo = wo_ref[0]

    xh = jnp.where(odd, x_ref[:, D_HEAD:], x_ref[:, :D_HEAD])
    q = jnp.where(odd, q_ref[:, D_HEAD:], q_ref[:, :D_HEAD])
    k = jnp.where(odd, k_ref[:, D_HEAD:], k_ref[:, :D_HEAD])

    idx = idx_ref[0]
    lane_e = jax.lax.broadcasted_iota(jnp.int32, (S_LEN, N_EXP), 1)

    cnt = jnp.sum((idx == lane_e).astype(f32), axis=0, keepdims=True)
    ema = ema_ref[0] * EMA_DECAY + cnt * ((1.0 - EMA_DECAY) / S_LEN)
    ema_out_ref[0] = ema

    scale = 1.0 / math.sqrt(D_HEAD)
    nb = S_LEN // S_BLK
    dims = (((1,), (1,)), ((), ()))
    lane_blk = jax.lax.broadcasted_iota(jnp.int32, (S_BLK, N_EXP * D_HEAD), 1) >> 6

    attn = jnp.zeros((S_LEN, D_HEAD), f32)
    for b in range(nb):
        lo, hi = b * S_BLK, (b + 1) * S_BLK
        qb = q[lo:hi, :]
        st = jax.lax.dot_general(k, qb, dims, preferred_element_type=f32)
        st = st * scale
        s_glob = b * S_BLK + jax.lax.broadcasted_iota(jnp.int32, (S_LEN, S_BLK), 1)
        t_row = jax.lax.broadcasted_iota(jnp.int32, (S_LEN, S_BLK), 0)
        st = st + jnp.where(t_row <= s_glob, 0.0, -1e9)
        cm = jnp.max(st, axis=0, keepdims=True)
        p = jnp.exp(st - cm)
        p = p / jnp.sum(p, axis=0, keepdims=True)

        xb = xh[lo:hi, :]
        xe = jnp.tile(xb, (1, N_EXP))
        xs = jnp.where(lane_blk == idx[lo:hi, :], xe, 0.0)
        vb = jax.lax.dot_general(xs, wv, (((1,), (0,)), ((), ())),
                                 preferred_element_type=f32)
        attn = attn + jax.lax.dot_general(p, vb, (((1,), (0,)), ((), ())),
                                          preferred_element_type=f32)

    lane128 = jax.lax.broadcasted_iota(jnp.int32, (S_BLK, 2 * D_HEAD), 1)
    mine = (lane128 >> 6) == (i & 1)
    for b in range(nb):
        lo, hi = b * S_BLK, (b + 1) * S_BLK
        ab = attn[lo:hi, :]
        ae = jnp.tile(ab, (1, N_EXP))
        as_ = jnp.where(lane_blk == idx[lo:hi, :], ae, 0.0)
        yb = jax.lax.dot_general(as_, wo, (((1,), (0,)), ((), ())),
                                 preferred_element_type=f32)
        y_ref[lo:hi, :] = jnp.where(mine, jnp.tile(yb, (1, 2)),
                                    y_ref[lo:hi, :])


@functools.partial(jax.jit, static_argnames=())
def kernel(x, mask, W_q, W_k, W_v, W_o, router_W, ema_counts):
    del mask
    B, S, D = x.shape
    h, E, d_h = N_HEAD, N_EXP, D_HEAD

    wv_flat = W_v.reshape(h, E * d_h, d_h)
    wo_flat = W_o.reshape(h, E * d_h, d_h)
    ema3 = ema_counts.reshape(h, 1, E)
    x2 = x.reshape(S, D)

    qk, gl = pl.pallas_call(
        _proj_kernel,
        grid=(1,),
        in_specs=[
            pl.BlockSpec((S, D), lambda i: (0, 0)),
            pl.BlockSpec((D, D), lambda i: (0, 0)),
            pl.BlockSpec((D, D), lambda i: (0, 0)),
            pl.BlockSpec((D, D), lambda i: (0, 0)),
        ],
        out_specs=[
            pl.BlockSpec((S, 2 * D), lambda i: (0, 0)),
            pl.BlockSpec((S, D), lambda i: (0, 0)),
        ],
        out_shape=[
            jax.ShapeDtypeStruct((S, 2 * D), jnp.float32),
            jax.ShapeDtypeStruct((S, D), jnp.float32),
        ],
    )(x2, W_q, W_k, router_W)

    route = functools.partial(
        pl.kernel,
        mesh=plsc.VectorSubcoreMesh(core_axis_name="c", subcore_axis_name="s"),
        out_type=jax.ShapeDtypeStruct((h * S,), jnp.int32),
        scratch_types=[
            pltpu.VMEM((_TOK_W * D,), jnp.float32),
            pltpu.VMEM((h * _TOK_W,), jnp.int32),
            pltpu.SemaphoreType.DMA,
        ],
    )(_route_sc)
    idx3 = route(gl.reshape(S * D)).reshape(h, S, 1)

    grid = (h,)
    y2, ema = pl.pallas_call(
        _head_kernel,
        grid=grid,
        in_specs=[
            pl.BlockSpec((S, 2 * d_h), lambda i: (0, i // 2)),
            pl.BlockSpec((S, 2 * d_h), lambda i: (0, i // 2)),
            pl.BlockSpec((S, 2 * d_h), lambda i: (0, D // (2 * d_h) + i // 2)),
            pl.BlockSpec((1, S, 1), lambda i: (i, 0, 0)),
            pl.BlockSpec((1, E * d_h, d_h), lambda i: (i, 0, 0)),
            pl.BlockSpec((1, E * d_h, d_h), lambda i: (i, 0, 0)),
            pl.BlockSpec((1, 1, E), lambda i: (i, 0, 0)),
        ],
        out_specs=[
            pl.BlockSpec((S, 2 * d_h), lambda i: (0, i // 2)),
            pl.BlockSpec((1, 1, E), lambda i: (i, 0, 0)),
        ],
        out_shape=[
            jax.ShapeDtypeStruct((S, D), jnp.float32),
            jax.ShapeDtypeStruct((h, 1, E), jnp.float32),
        ],
    )(x2, qk, qk, idx3, wv_flat, wo_flat, ema3)

    y = y2.reshape(1, S, D)
    ema2 = ema.reshape(h, E)
    lb_loss = (ema2 * ema2).sum() * (E * h) / jnp.square(ema2.sum() + 1e-9)
    return (y, lb_loss)
